# Initial kernel scaffold; baseline (speedup 1.0000x reference)
import jax
import jax.numpy as jnp
from jax.experimental import pallas as pl


def kernel(x, edge_index, edge_vals, weight):
    raise NotImplementedError("write your pallas kernel here")


# speedup vs baseline: 4.4049x; 4.4049x over previous
import functools

import jax
import jax.numpy as jnp
from jax import lax
from jax.experimental import pallas as pl
from jax.experimental.pallas import tpu as pltpu
from jax.experimental.pallas import tpu_sc as plsc

NUM_WORKERS = 32
BLK = 128
LANES = 16


def _make_sc_spmm(n_pad, d, nb):
    rows_per_tile = n_pad // 16
    mesh = plsc.VectorSubcoreMesh(core_axis_name="c", subcore_axis_name="s")

    @functools.partial(
        pl.kernel,
        out_type=jax.ShapeDtypeStruct((2, n_pad, d), jnp.float32),
        mesh=mesh,
        scratch_types=[
            pltpu.VMEM((nb, BLK), jnp.int32),
            pltpu.VMEM((nb, BLK), jnp.int32),
            pltpu.VMEM((nb, BLK), jnp.float32),
            pltpu.VMEM((BLK, d), jnp.float32),
            pltpu.VMEM_SHARED((n_pad, d), jnp.float32),
            pltpu.SemaphoreType.DMA,
        ],
    )
    def sc_spmm(row_hbm, col_hbm, val_hbm, x_hbm, zpart_hbm,
                row_buf, col_buf, val_buf, rows_v, zsh, gsem):
        c = lax.axis_index("c")
        s = lax.axis_index("s")
        w = c * 16 + s

        pltpu.sync_copy(row_hbm.at[w], row_buf)
        pltpu.sync_copy(col_hbm.at[w], col_buf)
        pltpu.sync_copy(val_hbm.at[w], val_buf)

        zeros = jnp.zeros((LANES,), jnp.float32)

        def zero_row(i, carry):
            for q in range(d // LANES):
                rows_v[i, pl.ds(q * LANES, LANES)] = zeros
            return carry

        lax.fori_loop(0, BLK, zero_row, 0)
        base = s * rows_per_tile
        off = 0
        while off < rows_per_tile:
            seg = min(BLK, rows_per_tile - off)
            pltpu.sync_copy(rows_v.at[pl.ds(0, seg)],
                            zsh.at[pl.ds(base + off, seg)])
            off += seg
        plsc.subcore_barrier()

        def do_block(j, carry):
            pltpu.async_copy(x_hbm.at[col_buf.at[j]], rows_v, gsem).wait()

            def scale_group(g, inner):
                vgrp = val_buf[j, pl.ds(g * LANES, LANES)]
                for i2 in range(LANES):
                    vsp = jnp.broadcast_to(vgrp[i2], (LANES,))
                    e_i = g * LANES + i2
                    for q in range(d // LANES):
                        sl = pl.ds(q * LANES, LANES)
                        rows_v[e_i, sl] = rows_v[e_i, sl] * vsp
                return inner

            lax.fori_loop(0, BLK // LANES, scale_group, 0)

            pltpu.sync_copy(rows_v, zsh.at[row_buf.at[j]], add=True)
            return carry

        lax.fori_loop(0, nb, do_block, 0)

        plsc.subcore_barrier()
        pltpu.sync_copy(zsh.at[pl.ds(base, rows_per_tile)],
                        zpart_hbm.at[c, pl.ds(base, rows_per_tile)])

    return sc_spmm


def _tc_matmul(zpart, weight, n):
    d = zpart.shape[2]
    d_out = weight.shape[1]
    bm = 1000

    def body(z_ref, w_ref, o_ref):
        z = z_ref[0] + z_ref[1]
        o_ref[...] = jnp.dot(z, w_ref[...],
                             preferred_element_type=jnp.float32,
                             precision=lax.Precision.HIGHEST)

    return pl.pallas_call(
        body,
        grid=(-(-n // bm),),
        in_specs=[
            pl.BlockSpec((2, bm, d), lambda i: (0, i, 0)),
            pl.BlockSpec((d, d_out), lambda i: (0, 0)),
        ],
        out_specs=pl.BlockSpec((bm, d_out), lambda i: (i, 0)),
        out_shape=jax.ShapeDtypeStruct((n, d_out), jnp.float32),
    )(zpart, weight)


def kernel(x, edge_index, edge_vals, weight):
    n, d = x.shape
    e = edge_index.shape[1]
    per = NUM_WORKERS * BLK
    nb = -(-e // per)
    e_pad = per * nb
    pad = e_pad - e

    row = edge_index[0].astype(jnp.int32)
    col = edge_index[1].astype(jnp.int32)
    val = edge_vals.astype(jnp.float32)
    if pad:
        row = jnp.concatenate([row, jnp.zeros((pad,), jnp.int32)])
        col = jnp.concatenate([col, jnp.zeros((pad,), jnp.int32)])
        val = jnp.concatenate([val, jnp.zeros((pad,), jnp.float32)])
    row = row.reshape(NUM_WORKERS, nb, BLK)
    col = col.reshape(NUM_WORKERS, nb, BLK)
    val = val.reshape(NUM_WORKERS, nb, BLK)

    n_pad = -(-n // BLK) * BLK
    zpart = _make_sc_spmm(n_pad, d, nb)(row, col, val, x)
    return _tc_matmul(zpart, weight, n)
